# baseline (speedup 1.0000x reference)
import functools

import jax
import jax.numpy as jnp
from jax import lax
from jax.experimental import pallas as pl
from jax.experimental.pallas import tpu as pltpu
from jax.experimental.pallas import tpu_sc as plsc

N_NODES = 10000
N_EDGES = 320000
D = 128
LANES = 16
NCORES = 2
NSUB = 16
CHUNK = 128
NCHUNK = 157
EPT = NCHUNK * CHUNK
E_PAD = NSUB * EPT
NPC = N_NODES // NCORES
ACC_ROWS = NPC + 8
ROWS_PT = 312
TAIL_ROWS = NPC - NSUB * ROWS_PT
ZTAIL = ACC_ROWS - NSUB * ROWS_PT
ZROWS = 24

_sc_mesh = plsc.VectorSubcoreMesh(core_axis_name="c", subcore_axis_name="s")


@functools.partial(
    pl.kernel,
    mesh=_sc_mesh,
    out_type=jax.ShapeDtypeStruct((N_NODES, D), jnp.float32),
    scratch_types=[
        pltpu.VMEM((NCHUNK, CHUNK), jnp.int32),
        pltpu.VMEM((NCHUNK, CHUNK), jnp.int32),
        pltpu.VMEM((NCHUNK, CHUNK), jnp.float32),
        pltpu.VMEM((CHUNK, D), jnp.float32),
        pltpu.VMEM((ZROWS, D), jnp.float32),
        pltpu.VMEM_SHARED((ACC_ROWS, D), jnp.float32),
        pltpu.SemaphoreType.DMA,
    ],
)
def _segment_sum(x_hbm, src_hbm, dst_hbm, w_hbm, out_hbm,
                 src_v, dst_v, w_v, rows_v, zero_v, acc_sh, sem):
    core = lax.axis_index("c")
    sub = lax.axis_index("s")

    pltpu.sync_copy(src_hbm.at[sub], src_v)
    pltpu.sync_copy(dst_hbm.at[sub], dst_v)
    pltpu.sync_copy(w_hbm.at[sub], w_v)

    base = jnp.full((LANES,), core * NPC, jnp.int32)
    npc = jnp.full((LANES,), NPC, jnp.int32)
    izero = jnp.zeros((LANES,), jnp.int32)

    def rbody(r, carry):
        for g in range(CHUNK // LANES):
            sl = pl.ds(g * LANES, LANES)
            t = dst_v[r, sl] - base
            keep = (t >= izero) & (t < npc)
            dst_v[r, sl] = jnp.where(keep, t, npc)
            src_v[r, sl] = jnp.where(keep, src_v[r, sl], izero)
        return carry

    lax.fori_loop(0, NCHUNK, rbody, 0)

    zvec = jnp.zeros((LANES,), jnp.float32)
    for r in range(ZROWS):
        for cg in range(D // LANES):
            zero_v[r, pl.ds(cg * LANES, LANES)] = zvec

    def zbody(i, carry):
        off = pl.multiple_of(sub * ROWS_PT + i * ZROWS, 8)
        pltpu.sync_copy(zero_v, acc_sh.at[pl.ds(off, ZROWS)])
        return carry

    lax.fori_loop(0, ROWS_PT // ZROWS, zbody, 0)

    @pl.when(sub == 0)
    def _zero_tail():
        pltpu.sync_copy(zero_v.at[pl.ds(0, ZTAIL)],
                        acc_sh.at[pl.ds(NSUB * ROWS_PT, ZTAIL)])

    plsc.subcore_barrier()

    def body(j, carry):
        pltpu.async_copy(x_hbm.at[src_v.at[j]], rows_v, sem).wait()
        for g in range(CHUNK // LANES):
            wvec = w_v[j, pl.ds(g * LANES, LANES)]
            for e in range(LANES):
                wb = wvec[e]
                row = g * LANES + e
                for cg in range(D // LANES):
                    sl = pl.ds(cg * LANES, LANES)
                    rows_v[row, sl] = rows_v[row, sl] * wb
        pltpu.sync_copy(rows_v, acc_sh.at[dst_v.at[j]], add=True)
        return carry

    lax.fori_loop(0, NCHUNK, body, 0)
    plsc.subcore_barrier()

    aoff = pl.multiple_of(sub * ROWS_PT, 8)
    ooff = pl.multiple_of(core * NPC + sub * ROWS_PT, 8)
    pltpu.sync_copy(acc_sh.at[pl.ds(aoff, ROWS_PT)],
                    out_hbm.at[pl.ds(ooff, ROWS_PT)])

    @pl.when(sub == 0)
    def _write_tail():
        toff = pl.multiple_of(core * NPC + NSUB * ROWS_PT, 8)
        pltpu.sync_copy(acc_sh.at[pl.ds(NSUB * ROWS_PT, TAIL_ROWS)],
                        out_hbm.at[pl.ds(toff, TAIL_ROWS)])


def _mlp_body(x_ref, agg_ref, wa_ref, ba_ref, wb_ref, bb_ref, g_ref, be_ref,
              o_ref, *, residual):
    h = x_ref[...] + agg_ref[...]
    a = jnp.maximum(
        jnp.dot(h, wa_ref[...], preferred_element_type=jnp.float32)
        + ba_ref[...], 0.0)
    t = (jnp.dot(a, wb_ref[...], preferred_element_type=jnp.float32)
         + bb_ref[...])
    mean = jnp.mean(t, axis=0, keepdims=True)
    var = jnp.mean(jnp.square(t - mean), axis=0, keepdims=True)
    y = (t - mean) * lax.rsqrt(var + 1e-5) * g_ref[...] + be_ref[...]
    y = jnp.maximum(y, 0.0)
    if residual:
        y = y + x_ref[...]
    o_ref[...] = y


def _mlp(x, agg, wa, ba, wb, bb, g, be, residual):
    body = functools.partial(_mlp_body, residual=residual)
    return pl.pallas_call(
        body,
        out_shape=jax.ShapeDtypeStruct((N_NODES, D), jnp.float32),
    )(x, agg, wa, ba.reshape(1, D), wb, bb.reshape(1, D),
      g.reshape(1, D), be.reshape(1, D))


def kernel(x, edge_index, edge_weight,
           W0a, b0a, W0b, b0b, g0, be0,
           W1a, b1a, W1b, b1b, g1, be1,
           W2a, b2a, W2b, b2b, g2, be2):
    ei = edge_index.astype(jnp.int32)
    pad = E_PAD - N_EDGES
    src = jnp.pad(ei[0], (0, pad)).reshape(NSUB, NCHUNK, CHUNK)
    dst = jnp.pad(ei[1], (0, pad)).reshape(NSUB, NCHUNK, CHUNK)
    w = jnp.pad(edge_weight, (0, pad)).reshape(NSUB, NCHUNK, CHUNK)

    def gin(h, wa, ba, wb, bb, g, be, residual):
        agg = _segment_sum(h, src, dst, w)
        return _mlp(h, agg, wa, ba, wb, bb, g, be, residual)

    h = gin(x, W0a, b0a, W0b, b0b, g0, be0, False)
    h = gin(h, W1a, b1a, W1b, b1b, g1, be1, True)
    return gin(h, W2a, b2a, W2b, b2b, g2, be2, True)


# speedup vs baseline: 22.9013x; 22.9013x over previous
import functools

import jax
import jax.numpy as jnp
from jax import lax
from jax.experimental import pallas as pl
from jax.experimental.pallas import tpu as pltpu
from jax.experimental.pallas import tpu_sc as plsc

N_NODES = 10000
N_EDGES = 320000
D = 128
LANES = 16
NCORES = 2
NSUB = 16
CHUNK = 112
NCHUNK = 180
EBUF = NCHUNK * CHUNK
E_PAD = NSUB * EBUF
NPC = N_NODES // NCORES
ACC_ROWS = NPC + 8
ROWS_PT = 312
TAIL_ROWS = NPC - NSUB * ROWS_PT
ZTAIL = ACC_ROWS - NSUB * ROWS_PT
ZROWS = 24

_sc_mesh = plsc.VectorSubcoreMesh(core_axis_name="c", subcore_axis_name="s")


@functools.partial(
    pl.kernel,
    mesh=_sc_mesh,
    out_type=jax.ShapeDtypeStruct((N_NODES, D), jnp.float32),
    scratch_types=[
        pltpu.VMEM((EBUF,), jnp.int32),
        pltpu.VMEM((EBUF,), jnp.int32),
        pltpu.VMEM((EBUF,), jnp.float32),
        pltpu.VMEM((2, CHUNK), jnp.int32),
        pltpu.VMEM((CHUNK, D), jnp.float32),
        pltpu.VMEM((CHUNK, D), jnp.float32),
        pltpu.VMEM_SHARED((ACC_ROWS, D), jnp.float32),
        pltpu.SemaphoreType.DMA,
        pltpu.SemaphoreType.DMA,
        pltpu.SemaphoreType.DMA,
        pltpu.SemaphoreType.DMA,
    ],
)
def _segment_sum(x_hbm, src_hbm, dst_hbm, w_hbm, out_hbm,
                 src_v, dst_v, w_v, dstage, rows0, rows1, acc_sh,
                 gs0, gs1, ss0, ss1):
    core = lax.axis_index("c")
    sub = lax.axis_index("s")
    rows = [rows0, rows1]
    gsem = [gs0, gs1]
    ssem = [ss0, ss1]

    pltpu.sync_copy(src_hbm.at[sub], src_v)
    pltpu.sync_copy(dst_hbm.at[sub], dst_v)
    pltpu.sync_copy(w_hbm.at[sub], w_v)

    base = jnp.full((LANES,), core * NPC, jnp.int32)
    npc = jnp.full((LANES,), NPC, jnp.int32)
    izero = jnp.zeros((LANES,), jnp.int32)
    fzero = jnp.zeros((LANES,), jnp.float32)

    def rbody(i, carry):
        sl = pl.ds(i * LANES, LANES)
        t = dst_v[sl] - base
        keep = (t >= izero) & (t < npc)
        dst_v[sl] = jnp.where(keep, t, npc)
        return carry

    lax.fori_loop(0, EBUF // LANES, rbody, 0)

    for r in range(ZROWS):
        for cg in range(D // LANES):
            rows0[r, pl.ds(cg * LANES, LANES)] = fzero

    def zbody(i, carry):
        off = pl.multiple_of(sub * ROWS_PT + i * ZROWS, 8)
        pltpu.sync_copy(rows0.at[pl.ds(0, ZROWS)],
                        acc_sh.at[pl.ds(off, ZROWS)])
        return carry

    lax.fori_loop(0, ROWS_PT // ZROWS, zbody, 0)

    @pl.when(sub == 0)
    def _zero_tail():
        pltpu.sync_copy(rows0.at[pl.ds(0, ZTAIL)],
                        acc_sh.at[pl.ds(NSUB * ROWS_PT, ZTAIL)])

    plsc.subcore_barrier()

    def gather_start(j, b):
        pltpu.async_copy(x_hbm.at[src_v.at[pl.ds(j * CHUNK, CHUNK)]],
                         rows[b], gsem[b])

    def gather_wait(j, b):
        pltpu.make_async_copy(x_hbm.at[src_v.at[pl.ds(j * CHUNK, CHUNK)]],
                              rows[b], gsem[b]).wait()

    def weight(j, b):
        rv = rows[b]
        eb = j * CHUNK
        for k in range(CHUNK // LANES):
            dstage[b, pl.ds(k * LANES, LANES)] = \
                dst_v[pl.ds(eb + k * LANES, LANES)]
        for g in range(CHUNK // LANES):
            wvec = w_v[pl.ds(eb + g * LANES, LANES)]
            for e in range(LANES):
                wb = wvec[e]
                row = g * LANES + e
                for cg in range(D // LANES):
                    sl = pl.ds(cg * LANES, LANES)
                    rv[row, sl] = rv[row, sl] * wb

    def scatter_start(b):
        pltpu.async_copy(rows[b], acc_sh.at[dstage.at[b]], ssem[b],
                         add=True)

    def scatter_wait(b):
        pltpu.make_async_copy(rows[b], acc_sh.at[dstage.at[b]],
                              ssem[b]).wait()

    gather_start(0, 0)

    def body(i, carry):
        j0 = 2 * i

        @pl.when(i > 0)
        def _drain1():
            scatter_wait(1)

        gather_start(j0 + 1, 1)
        gather_wait(j0, 0)
        weight(j0, 0)
        scatter_start(0)
        gather_wait(j0 + 1, 1)
        weight(j0 + 1, 1)
        scatter_wait(0)

        @pl.when(i < NCHUNK // 2 - 1)
        def _next():
            gather_start(j0 + 2, 0)

        scatter_start(1)
        return carry

    lax.fori_loop(0, NCHUNK // 2, body, 0)
    scatter_wait(1)
    plsc.subcore_barrier()

    aoff = pl.multiple_of(sub * ROWS_PT, 8)
    ooff = pl.multiple_of(core * NPC + sub * ROWS_PT, 8)
    pltpu.sync_copy(acc_sh.at[pl.ds(aoff, ROWS_PT)],
                    out_hbm.at[pl.ds(ooff, ROWS_PT)])

    @pl.when(sub == 0)
    def _write_tail():
        toff = pl.multiple_of(core * NPC + NSUB * ROWS_PT, 8)
        pltpu.sync_copy(acc_sh.at[pl.ds(NSUB * ROWS_PT, TAIL_ROWS)],
                        out_hbm.at[pl.ds(toff, TAIL_ROWS)])


def _mlp_body(x_ref, agg_ref, wa_ref, ba_ref, wb_ref, bb_ref, g_ref, be_ref,
              o_ref, *, residual):
    h = x_ref[...] + agg_ref[...]
    a = jnp.maximum(
        jnp.dot(h, wa_ref[...], preferred_element_type=jnp.float32)
        + ba_ref[...], 0.0)
    t = (jnp.dot(a, wb_ref[...], preferred_element_type=jnp.float32)
         + bb_ref[...])
    mean = jnp.mean(t, axis=0, keepdims=True)
    var = jnp.mean(jnp.square(t - mean), axis=0, keepdims=True)
    y = (t - mean) * lax.rsqrt(var + 1e-5) * g_ref[...] + be_ref[...]
    y = jnp.maximum(y, 0.0)
    if residual:
        y = y + x_ref[...]
    o_ref[...] = y


def _mlp(x, agg, wa, ba, wb, bb, g, be, residual):
    body = functools.partial(_mlp_body, residual=residual)
    return pl.pallas_call(
        body,
        out_shape=jax.ShapeDtypeStruct((N_NODES, D), jnp.float32),
    )(x, agg, wa, ba.reshape(1, D), wb, bb.reshape(1, D),
      g.reshape(1, D), be.reshape(1, D))


def kernel(x, edge_index, edge_weight,
           W0a, b0a, W0b, b0b, g0, be0,
           W1a, b1a, W1b, b1b, g1, be1,
           W2a, b2a, W2b, b2b, g2, be2):
    ei = edge_index.astype(jnp.int32)
    pad = E_PAD - N_EDGES
    src = jnp.pad(ei[0], (0, pad)).reshape(NSUB, EBUF)
    dst = jnp.pad(ei[1], (0, pad), constant_values=N_NODES).reshape(
        NSUB, EBUF)
    w = jnp.pad(edge_weight, (0, pad)).reshape(NSUB, EBUF)

    def gin(h, wa, ba, wb, bb, g, be, residual):
        agg = _segment_sum(h, src, dst, w)
        return _mlp(h, agg, wa, ba, wb, bb, g, be, residual)

    h = gin(x, W0a, b0a, W0b, b0b, g0, be0, False)
    h = gin(h, W1a, b1a, W1b, b1b, g1, be1, True)
    return gin(h, W2a, b2a, W2b, b2b, g2, be2, True)


# speedup vs baseline: 22.9694x; 1.0030x over previous
import functools

import jax
import jax.numpy as jnp
from jax import lax
from jax.experimental import pallas as pl
from jax.experimental.pallas import tpu as pltpu
from jax.experimental.pallas import tpu_sc as plsc

N_NODES = 10000
N_EDGES = 320000
D = 128
LANES = 16
NCORES = 2
NSUB = 16
CHUNK = 128
NCHUNK = 157
EPT = NCHUNK * CHUNK
E_PAD = NSUB * EPT
NPC = N_NODES // NCORES
ACC_ROWS = NPC + 8
ROWS_PT = 312
TAIL_ROWS = NPC - NSUB * ROWS_PT
ZTAIL = ACC_ROWS - NSUB * ROWS_PT
ZROWS = 24

_sc_mesh = plsc.VectorSubcoreMesh(core_axis_name="c", subcore_axis_name="s")


@functools.partial(
    pl.kernel,
    mesh=_sc_mesh,
    out_type=jax.ShapeDtypeStruct((N_NODES, D), jnp.float32),
    scratch_types=[
        pltpu.VMEM((NCHUNK, CHUNK), jnp.int32),
        pltpu.VMEM((NCHUNK, CHUNK), jnp.int32),
        pltpu.VMEM((NCHUNK, CHUNK), jnp.float32),
        pltpu.VMEM((CHUNK, D), jnp.float32),
        pltpu.VMEM((ZROWS, D), jnp.float32),
        pltpu.VMEM_SHARED((ACC_ROWS, D), jnp.float32),
        pltpu.SemaphoreType.DMA,
    ],
)
def _segment_sum(x_hbm, src_hbm, dst_hbm, w_hbm, out_hbm,
                 src_v, dst_v, w_v, rows_v, zero_v, acc_sh, sem):
    core = lax.axis_index("c")
    sub = lax.axis_index("s")

    pltpu.sync_copy(src_hbm.at[sub], src_v)
    pltpu.sync_copy(dst_hbm.at[sub], dst_v)
    pltpu.sync_copy(w_hbm.at[sub], w_v)

    base = jnp.full((LANES,), core * NPC, jnp.int32)
    npc = jnp.full((LANES,), NPC, jnp.int32)
    izero = jnp.zeros((LANES,), jnp.int32)

    def rbody(r, carry):
        for g in range(CHUNK // LANES):
            sl = pl.ds(g * LANES, LANES)
            t = dst_v[r, sl] - base
            keep = (t >= izero) & (t < npc)
            dst_v[r, sl] = jnp.where(keep, t, npc)
        return carry

    lax.fori_loop(0, NCHUNK, rbody, 0)

    zvec = jnp.zeros((LANES,), jnp.float32)
    for r in range(ZROWS):
        for cg in range(D // LANES):
            zero_v[r, pl.ds(cg * LANES, LANES)] = zvec

    def zbody(i, carry):
        off = pl.multiple_of(sub * ROWS_PT + i * ZROWS, 8)
        pltpu.sync_copy(zero_v, acc_sh.at[pl.ds(off, ZROWS)])
        return carry

    lax.fori_loop(0, ROWS_PT // ZROWS, zbody, 0)

    @pl.when(sub == 0)
    def _zero_tail():
        pltpu.sync_copy(zero_v.at[pl.ds(0, ZTAIL)],
                        acc_sh.at[pl.ds(NSUB * ROWS_PT, ZTAIL)])

    plsc.subcore_barrier()

    def body(j, carry):
        pltpu.async_copy(x_hbm.at[src_v.at[j]], rows_v, sem).wait()
        for g in range(CHUNK // LANES):
            wvec = w_v[j, pl.ds(g * LANES, LANES)]
            for e in range(LANES):
                wb = wvec[e]
                row = g * LANES + e
                for cg in range(D // LANES):
                    sl = pl.ds(cg * LANES, LANES)
                    rows_v[row, sl] = rows_v[row, sl] * wb
        pltpu.sync_copy(rows_v, acc_sh.at[dst_v.at[j]], add=True)
        return carry

    lax.fori_loop(0, NCHUNK, body, 0)
    plsc.subcore_barrier()

    aoff = pl.multiple_of(sub * ROWS_PT, 8)
    ooff = pl.multiple_of(core * NPC + sub * ROWS_PT, 8)
    pltpu.sync_copy(acc_sh.at[pl.ds(aoff, ROWS_PT)],
                    out_hbm.at[pl.ds(ooff, ROWS_PT)])

    @pl.when(sub == 0)
    def _write_tail():
        toff = pl.multiple_of(core * NPC + NSUB * ROWS_PT, 8)
        pltpu.sync_copy(acc_sh.at[pl.ds(NSUB * ROWS_PT, TAIL_ROWS)],
                        out_hbm.at[pl.ds(toff, TAIL_ROWS)])


def _mlp_body(x_ref, agg_ref, wa_ref, ba_ref, wb_ref, bb_ref, g_ref, be_ref,
              o_ref, *, residual):
    h = x_ref[...] + agg_ref[...]
    a = jnp.maximum(
        jnp.dot(h, wa_ref[...], preferred_element_type=jnp.float32)
        + ba_ref[...], 0.0)
    t = (jnp.dot(a, wb_ref[...], preferred_element_type=jnp.float32)
         + bb_ref[...])
    mean = jnp.mean(t, axis=0, keepdims=True)
    var = jnp.mean(jnp.square(t - mean), axis=0, keepdims=True)
    y = (t - mean) * lax.rsqrt(var + 1e-5) * g_ref[...] + be_ref[...]
    y = jnp.maximum(y, 0.0)
    if residual:
        y = y + x_ref[...]
    o_ref[...] = y


def _mlp(x, agg, wa, ba, wb, bb, g, be, residual):
    body = functools.partial(_mlp_body, residual=residual)
    return pl.pallas_call(
        body,
        out_shape=jax.ShapeDtypeStruct((N_NODES, D), jnp.float32),
    )(x, agg, wa, ba.reshape(1, D), wb, bb.reshape(1, D),
      g.reshape(1, D), be.reshape(1, D))


def kernel(x, edge_index, edge_weight,
           W0a, b0a, W0b, b0b, g0, be0,
           W1a, b1a, W1b, b1b, g1, be1,
           W2a, b2a, W2b, b2b, g2, be2):
    ei = edge_index.astype(jnp.int32)
    pad = E_PAD - N_EDGES
    src = jnp.pad(ei[0], (0, pad)).reshape(NSUB, NCHUNK, CHUNK)
    dst = jnp.pad(ei[1], (0, pad)).reshape(NSUB, NCHUNK, CHUNK)
    w = jnp.pad(edge_weight, (0, pad)).reshape(NSUB, NCHUNK, CHUNK)

    def gin(h, wa, ba, wb, bb, g, be, residual):
        agg = _segment_sum(h, src, dst, w)
        return _mlp(h, agg, wa, ba, wb, bb, g, be, residual)

    h = gin(x, W0a, b0a, W0b, b0b, g0, be0, False)
    h = gin(h, W1a, b1a, W1b, b1b, g1, be1, True)
    return gin(h, W2a, b2a, W2b, b2b, g2, be2, True)
